# Initial kernel scaffold; baseline (speedup 1.0000x reference)
import jax
import jax.numpy as jnp
from jax.experimental import pallas as pl


def kernel(image, defocus):
    raise NotImplementedError("write your pallas kernel here")


# speedup vs baseline: 896.8055x; 896.8055x over previous
import jax
import jax.numpy as jnp
from jax.experimental import pallas as pl
from jax.experimental.pallas import tpu as pltpu

H = 512
W = 512
OFF = 8
K = 2 * OFF + 1
T = 64
L = 11.541560327111707
SENTINEL = -1000.0


def _body(rpad_ref, impad_ref, bokeh_ref, dil_ref, acc_b, acc_w, acc_d):
    i = pl.program_id(0)
    k = pl.program_id(1)

    @pl.when(k == 0)
    def _init():
        acc_w[...] = jnp.zeros((T, W), jnp.float32)
        acc_b[...] = jnp.zeros((3, T, W), jnp.float32)
        acc_d[...] = jnp.zeros((T, W), jnp.float32)

    row0 = i * T
    rb = pltpu.roll(rpad_ref[pl.ds(row0, T + 16), :], (T + 16) - k, axis=0)[:T]
    im0 = pltpu.roll(impad_ref[0, pl.ds(row0, T + 16), :], (T + 16) - k, axis=0)[:T]
    im1 = pltpu.roll(impad_ref[1, pl.ds(row0, T + 16), :], (T + 16) - k, axis=0)[:T]
    im2 = pltpu.roll(impad_ref[2, pl.ds(row0, T + 16), :], (T + 16) - k, axis=0)[:T]
    vb = rb * rb + 0.2
    fb = jnp.floor(rb)

    dyf = (k - OFF).astype(jnp.float32)
    ady = jnp.abs(dyf)
    dy2 = dyf * dyf

    wl = jnp.zeros((T, W), jnp.float32)
    b0 = jnp.zeros((T, W), jnp.float32)
    b1 = jnp.zeros((T, W), jnp.float32)
    b2 = jnp.zeros((T, W), jnp.float32)
    dl = jnp.zeros((T, W), jnp.float32)
    for dx in range(-OFF, OFF + 1):
        sl = slice(OFF + dx, OFF + dx + W)
        rs = rb[:, sl]
        vs = vb[:, sl]
        fs = fb[:, sl]
        dist = jnp.sqrt(dy2 + jnp.float32(dx * dx))
        m1 = jnp.maximum(ady, jnp.float32(abs(dx))) - 1.0
        t = jnp.exp2((dist - rs) * L)
        w = 1.0 / ((1.0 + t) * vs)
        w = jnp.where(rs >= m1, w, 0.0)
        wl += w
        b0 += w * im0[:, sl]
        b1 += w * im1[:, sl]
        b2 += w * im2[:, sl]
        dl = jnp.maximum(dl, jnp.where(rs >= dist, fs, 0.0))

    acc_w[...] += wl
    acc_b[0] += b0
    acc_b[1] += b1
    acc_b[2] += b2
    acc_d[...] = jnp.maximum(acc_d[...], dl)

    @pl.when(k == K - 1)
    def _finalize():
        inv = 1.0 / acc_w[...]
        bokeh_ref[0] = acc_b[0] * inv
        bokeh_ref[1] = acc_b[1] * inv
        bokeh_ref[2] = acc_b[2] * inv
        dorig = rpad_ref[pl.ds(i * T + OFF, T), :][:, OFF:OFF + W]
        dil_ref[...] = jnp.maximum(dorig, acc_d[...]).astype(jnp.int32)


def kernel(image, defocus):
    d = defocus[0, 0]
    img = image[0]
    rpad = jnp.pad(d, OFF, constant_values=SENTINEL)
    impad = jnp.pad(img, ((0, 0), (OFF, OFF), (OFF, OFF)))

    grid = (H // T, K)
    bokeh, dil = pl.pallas_call(
        _body,
        grid=grid,
        in_specs=[
            pl.BlockSpec((H + 16, W + 16), lambda i, k: (0, 0)),
            pl.BlockSpec((3, H + 16, W + 16), lambda i, k: (0, 0, 0)),
        ],
        out_specs=[
            pl.BlockSpec((3, T, W), lambda i, k: (0, i, 0)),
            pl.BlockSpec((T, W), lambda i, k: (i, 0)),
        ],
        out_shape=[
            jax.ShapeDtypeStruct((3, H, W), jnp.float32),
            jax.ShapeDtypeStruct((H, W), jnp.int32),
        ],
        scratch_shapes=[
            pltpu.VMEM((3, T, W), jnp.float32),
            pltpu.VMEM((T, W), jnp.float32),
            pltpu.VMEM((T, W), jnp.float32),
        ],
    )(rpad, impad)
    return bokeh.reshape(1, 3, H, W), dil.reshape(1, 1, H, W)


# speedup vs baseline: 1488.1132x; 1.6593x over previous
import jax
import jax.numpy as jnp
from jax.experimental import pallas as pl
from jax.experimental.pallas import tpu as pltpu

H = 512
W = 512
OFF = 8
K = 2 * OFF + 1
T = 64
L = 11.541560327111707
SENTINEL = -1000.0


def _body(rpad_ref, impad_ref, d_ref, bokeh_ref, dil_ref, acc_b, acc_w, acc_d):
    i = pl.program_id(0)
    k = pl.program_id(1)

    @pl.when(k == 0)
    def _init():
        acc_w[...] = jnp.zeros((T, W), jnp.float32)
        acc_b[...] = jnp.zeros((3, T, W), jnp.float32)
        acc_d[...] = jnp.zeros((T, W), jnp.float32)

    row0 = i * T
    rb = pltpu.roll(rpad_ref[pl.ds(row0, T + 16), :], (T + 16) - k, axis=0)[:T]
    im0 = pltpu.roll(impad_ref[0, pl.ds(row0, T + 16), :], (T + 16) - k, axis=0)[:T]
    im1 = pltpu.roll(impad_ref[1, pl.ds(row0, T + 16), :], (T + 16) - k, axis=0)[:T]
    im2 = pltpu.roll(impad_ref[2, pl.ds(row0, T + 16), :], (T + 16) - k, axis=0)[:T]
    vb = rb * rb + 0.2
    fb = jnp.floor(rb)

    dyf = (k - OFF).astype(jnp.float32)
    dy2 = dyf * dyf

    wl = jnp.zeros((T, W), jnp.float32)
    b0 = jnp.zeros((T, W), jnp.float32)
    b1 = jnp.zeros((T, W), jnp.float32)
    b2 = jnp.zeros((T, W), jnp.float32)
    dl = jnp.zeros((T, W), jnp.bfloat16)
    bf = jnp.bfloat16
    f32 = jnp.float32
    for adx in range(OFF + 1):
        dist = jnp.sqrt(dy2 + jnp.float32(adx * adx))
        t = jnp.exp2(rb * (-L) + (L * dist))
        wf = pl.reciprocal(t * vb + vb, approx=True)
        sls = [slice(OFF + dx, OFF + dx + W)
               for dx in ((adx, -adx) if adx else (0,))]
        wf16 = wf.astype(bf)
        for sl in sls:
            wl += wf16[:, sl].astype(f32)
        p016 = (wf * im0).astype(bf)
        for sl in sls:
            b0 += p016[:, sl].astype(f32)
        p116 = (wf * im1).astype(bf)
        for sl in sls:
            b1 += p116[:, sl].astype(f32)
        p216 = (wf * im2).astype(bf)
        for sl in sls:
            b2 += p216[:, sl].astype(f32)
        df16 = jnp.where(rb >= dist, fb, 0.0).astype(bf)
        for sl in sls:
            dl = jnp.maximum(dl, df16[:, sl])

    acc_w[...] += wl
    acc_b[0] += b0
    acc_b[1] += b1
    acc_b[2] += b2
    acc_d[...] = jnp.maximum(acc_d[...], dl.astype(jnp.float32))

    @pl.when(k == K - 1)
    def _finalize():
        inv = pl.reciprocal(acc_w[...], approx=True)
        bokeh_ref[0] = acc_b[0] * inv
        bokeh_ref[1] = acc_b[1] * inv
        bokeh_ref[2] = acc_b[2] * inv
        dil_ref[...] = jnp.maximum(d_ref[...], acc_d[...]).astype(jnp.int32)


def kernel(image, defocus):
    d = defocus[0, 0]
    img = image[0]
    rpad = jnp.pad(d, OFF, constant_values=SENTINEL)
    impad = jnp.pad(img, ((0, 0), (OFF, OFF), (OFF, OFF)))

    grid = (H // T, K)
    bokeh, dil = pl.pallas_call(
        _body,
        grid=grid,
        in_specs=[
            pl.BlockSpec((H + 16, W + 16), lambda i, k: (0, 0)),
            pl.BlockSpec((3, H + 16, W + 16), lambda i, k: (0, 0, 0)),
            pl.BlockSpec((T, W), lambda i, k: (i, 0)),
        ],
        out_specs=[
            pl.BlockSpec((3, T, W), lambda i, k: (0, i, 0)),
            pl.BlockSpec((T, W), lambda i, k: (i, 0)),
        ],
        out_shape=[
            jax.ShapeDtypeStruct((3, H, W), jnp.float32),
            jax.ShapeDtypeStruct((H, W), jnp.int32),
        ],
        scratch_shapes=[
            pltpu.VMEM((3, T, W), jnp.float32),
            pltpu.VMEM((T, W), jnp.float32),
            pltpu.VMEM((T, W), jnp.float32),
        ],
    )(rpad, impad, d)
    return bokeh.reshape(1, 3, H, W), dil.reshape(1, 1, H, W)
